# baseline (speedup 1.0000x reference)
import functools

import jax
import jax.numpy as jnp
from jax import lax
from jax.experimental import pallas as pl
from jax.experimental.pallas import tpu as pltpu
from jax.experimental.pallas import tpu_sc as plsc

N = 10000
D = 128
E = 320000
TAU = 2.0
V_TH = 1.0
BN_EPS = 1e-5

N_PAD = 10240
EPT = E // 32
C = 168
NWIN_FULL = 59
TAIL = EPT - NWIN_FULL * C
WPB = 8
NBLK = 8
BLKW = 2 * WPB * C
TPT = NBLK * BLKW + 2 * TAIL
CD = 1000
NWD = EPT // CD
RPT = N_PAD // 16
assert NWIN_FULL % 2 == 1 and 0 < TAIL <= C and TAIL % 8 == 0 and C % 8 == 0


def _deg_body(dst_hbm, ones_hbm, zeros_hbm, out_hbm, idx_v, ones_v, acc):
    c = lax.axis_index("c")
    s = lax.axis_index("s")
    pltpu.sync_copy(zeros_hbm.at[pl.ds(s * RPT, RPT)], acc.at[pl.ds(s * RPT, RPT)])
    pltpu.sync_copy(ones_hbm, ones_v)
    pltpu.sync_copy(dst_hbm.at[pl.ds((c * 16 + s) * EPT, EPT)], idx_v)
    plsc.subcore_barrier()

    def body(w, carry):
        pltpu.sync_copy(ones_v, acc.at[idx_v.at[pl.ds(w * CD, CD)]], add=True)
        return carry

    lax.fori_loop(0, NWD, body, 0)
    plsc.subcore_barrier()
    pltpu.sync_copy(acc.at[pl.ds(s * RPT, RPT)],
                    out_hbm.at[pl.ds(c * N_PAD + s * RPT, RPT)])


def _spmm_body(idx_hbm, g_hbm, zeros_hbm, out_hbm,
               idx_v, rows0, rows1, acc, sem0, sem1, sem_i):
    c = lax.axis_index("c")
    s = lax.axis_index("s")
    base = (c * 16 + s) * TPT
    rows = (rows0, rows1)
    sems = (sem0, sem1)

    def src_slice(w):
        p = (w // WPB) % 2
        return idx_v.at[pl.ds(p * BLKW + (w % WPB) * C, C)]

    def dst_slice(w):
        p = (w // WPB) % 2
        return idx_v.at[pl.ds(p * BLKW + WPB * C + (w % WPB) * C, C)]

    def blk_copy(w):
        blk = w // WPB
        return pltpu.make_async_copy(
            idx_hbm.at[pl.ds(base + blk * BLKW, BLKW)],
            idx_v.at[pl.ds((blk % 2) * BLKW, BLKW)], sem_i)

    def idx_gather(w, b):
        pltpu.async_copy(g_hbm.at[src_slice(w)], rows[b], sems[b])

    def gwait(w, b):
        pltpu.make_async_copy(g_hbm.at[src_slice(w)], rows[b], sems[b]).wait()

    def scat(w, b):
        pltpu.sync_copy(rows[b], acc.at[dst_slice(w)], add=True)

    @pl.when(c == 0)
    def _():
        pltpu.sync_copy(g_hbm.at[pl.ds(s * RPT, RPT)], acc.at[pl.ds(s * RPT, RPT)])

    @pl.when(c != 0)
    def _():
        pltpu.sync_copy(zeros_hbm.at[pl.ds(s * RPT, RPT)], acc.at[pl.ds(s * RPT, RPT)])

    blk_copy(0).start()
    blk_copy(0).wait()
    idx_gather(0, 0)
    plsc.subcore_barrier()

    def body(k, carry):
        w0 = 2 * k
        nb = (w0 + 2) % WPB == 0

        @pl.when(nb)
        def _():
            blk_copy(w0 + 2).start()

        gwait(w0, 0)
        idx_gather(w0 + 1, 1)
        scat(w0, 0)
        gwait(w0 + 1, 1)

        @pl.when(nb)
        def _():
            blk_copy(w0 + 2).wait()

        idx_gather(w0 + 2, 0)
        scat(w0 + 1, 1)
        return carry

    lax.fori_loop(0, NWIN_FULL // 2 - 1, body, 0)
    wl = NWIN_FULL - 3
    gwait(wl, 0)
    idx_gather(wl + 1, 1)
    scat(wl, 0)
    gwait(wl + 1, 1)
    idx_gather(wl + 2, 0)
    scat(wl + 1, 1)
    gwait(wl + 2, 0)
    toff = base + NBLK * BLKW
    tv = 2 * BLKW
    pltpu.sync_copy(idx_hbm.at[pl.ds(toff, 2 * TAIL)],
                    idx_v.at[pl.ds(tv, 2 * TAIL)])
    pltpu.async_copy(g_hbm.at[idx_v.at[pl.ds(tv, TAIL)]],
                     rows1.at[pl.ds(0, TAIL)], sem1)
    scat(wl + 2, 0)
    pltpu.make_async_copy(g_hbm.at[idx_v.at[pl.ds(tv, TAIL)]],
                          rows1.at[pl.ds(0, TAIL)], sem1).wait()
    pltpu.sync_copy(rows1.at[pl.ds(0, TAIL)],
                    acc.at[idx_v.at[pl.ds(tv + TAIL, TAIL)]], add=True)
    plsc.subcore_barrier()
    pltpu.sync_copy(acc.at[pl.ds(s * RPT, RPT)],
                    out_hbm.at[pl.ds(c * N_PAD + s * RPT, RPT)])


@functools.cache
def _sc_kernels():
    mesh = plsc.VectorSubcoreMesh(core_axis_name="c", subcore_axis_name="s")
    deg = pl.kernel(
        _deg_body,
        out_type=jax.ShapeDtypeStruct((2 * N_PAD,), jnp.float32),
        mesh=mesh,
        scratch_types=[
            pltpu.VMEM((EPT,), jnp.int32),
            pltpu.VMEM((CD,), jnp.float32),
            pltpu.VMEM_SHARED((N_PAD,), jnp.float32),
        ],
    )
    spmm = pl.kernel(
        _spmm_body,
        out_type=jax.ShapeDtypeStruct((2 * N_PAD, D), jnp.float32),
        mesh=mesh,
        scratch_types=[
            pltpu.VMEM((2 * BLKW + 2 * TAIL,), jnp.int32),
            pltpu.VMEM((C, D), jnp.float32),
            pltpu.VMEM((C, D), jnp.float32),
            pltpu.VMEM_SHARED((N_PAD, D), jnp.float32),
            pltpu.SemaphoreType.DMA,
            pltpu.SemaphoreType.DMA,
            pltpu.SemaphoreType.DMA,
        ],
    )
    return deg, spmm


def _pack_edges(src, dst):
    full = NWIN_FULL * C
    pad_w = NBLK * WPB - NWIN_FULL
    s2 = src.reshape(32, EPT)
    d2 = dst.reshape(32, EPT)

    def blocks(a):
        w = a[:, :full].reshape(32, NWIN_FULL, C)
        w = jnp.pad(w, ((0, 0), (0, pad_w), (0, 0)))
        return w.reshape(32, NBLK, WPB * C)

    blk = jnp.stack([blocks(s2), blocks(d2)], axis=2)
    tails = jnp.concatenate([s2[:, full:], d2[:, full:]], axis=1)
    return jnp.concatenate(
        [blk.reshape(32, NBLK * BLKW), tails], axis=1).reshape(-1)


R = 512
GB = N_PAD // R


def _prep_body(x_ref, da_ref, db_ref, g0_ref, dis_ref, inv_ref):
    deg = da_ref[...] + db_ref[...] + 1.0
    dis = jnp.where(deg > 0, lax.rsqrt(jnp.maximum(deg, 1e-12)), 0.0)
    dis_ref[...] = dis
    inv_ref[...] = dis * dis
    g0_ref[...] = x_ref[...] * dis


def _rescale_body(sa_ref, sb_ref, inv_ref, g1_ref):
    g1_ref[...] = (sa_ref[...] + sb_ref[...]) * inv_ref[...]


def _mmba_body(sa_ref, sb_ref, dis_ref, w_ref, gam_ref, bet_ref, o_ref,
               y_scr, cs_ref, cq_ref):
    i = pl.program_id(0)

    @pl.when(i == 0)
    def _():
        cs_ref[...] = jnp.zeros_like(cs_ref)
        cq_ref[...] = jnp.zeros_like(cq_ref)

    @pl.when(i < GB)
    def _():
        st = (sa_ref[...] + sb_ref[...]) * dis_ref[...]
        y = lax.dot_general(st, w_ref[...], (((1,), (1,)), ((), ())),
                            preferred_element_type=jnp.float32)
        y_scr[pl.ds(i * R, R), :] = y
        cs_ref[...] += jnp.sum(y.reshape(R // 8, 8, D), axis=0)
        cq_ref[...] += jnp.sum((y * y).reshape(R // 8, 8, D), axis=0)

    @pl.when(i >= GB)
    def _():
        j = i - GB
        invn = 1.0 / float(N)
        mean = jnp.sum(cs_ref[...], axis=0, keepdims=True) * invn
        ex2 = jnp.sum(cq_ref[...], axis=0, keepdims=True) * invn
        var = ex2 - mean * mean
        r = lax.rsqrt(var + BN_EPS)
        y = y_scr[pl.ds(j * R, R), :]
        hbn = (y - mean) * (r * gam_ref[0:1, :]) + bet_ref[0:1, :]
        v = hbn * (1.0 / TAU)
        o_ref[...] = jnp.where(v >= V_TH, 1.0, 0.0)


_blk_rd = lambda off=0: pl.BlockSpec((R, D), lambda i, o=off: (i + o, 0))
_blk_r1 = lambda off=0: pl.BlockSpec((R, 1), lambda i, o=off: (i + o, 0))
_blk_s8 = pl.BlockSpec((8, D), lambda i: (0, 0))

_prep = pl.pallas_call(
    _prep_body,
    grid=(GB,),
    in_specs=[_blk_rd(), _blk_r1(), _blk_r1(GB)],
    out_specs=[_blk_rd(), _blk_r1(), _blk_r1()],
    out_shape=[
        jax.ShapeDtypeStruct((N_PAD, D), jnp.float32),
        jax.ShapeDtypeStruct((N_PAD, 1), jnp.float32),
        jax.ShapeDtypeStruct((N_PAD, 1), jnp.float32),
    ],
)

_rescale = pl.pallas_call(
    _rescale_body,
    grid=(GB,),
    in_specs=[_blk_rd(), _blk_rd(GB), _blk_r1()],
    out_specs=_blk_rd(),
    out_shape=jax.ShapeDtypeStruct((N_PAD, D), jnp.float32),
)

_blk_in = lambda off=0: pl.BlockSpec(
    (R, D), lambda i, o=off: (jnp.minimum(i, GB - 1) + o, 0))
_blk_in1 = pl.BlockSpec((R, 1), lambda i: (jnp.minimum(i, GB - 1), 0))

_mmba = pl.pallas_call(
    _mmba_body,
    grid=(2 * GB,),
    in_specs=[_blk_in(), _blk_in(GB), _blk_in1,
              pl.BlockSpec((D, D), lambda i: (0, 0)), _blk_s8, _blk_s8],
    out_specs=pl.BlockSpec((R, D), lambda i: (jnp.maximum(i - GB, 0), 0)),
    out_shape=jax.ShapeDtypeStruct((N, D), jnp.float32),
    scratch_shapes=[
        pltpu.VMEM((N_PAD, D), jnp.float32),
        pltpu.VMEM((8, D), jnp.float32),
        pltpu.VMEM((8, D), jnp.float32),
    ],
)


def kernel(x, edge_index, W, b, gamma, beta):
    f32 = jnp.float32
    src = edge_index[0]
    dst = edge_index[1]
    x_pad = jnp.pad(x, ((0, N_PAD - N), (0, 0)))
    zeros2d = jnp.zeros((N_PAD, D), f32)
    zeros1d = jnp.zeros((N_PAD,), f32)
    ones_cd = jnp.ones((CD,), f32)

    _deg_kernel, _spmm_kernel = _sc_kernels()
    packed = _pack_edges(src, dst)
    degs = _deg_kernel(dst, ones_cd, zeros1d).reshape(2 * N_PAD, 1)
    g0, dis, inv = _prep(x_pad, degs, degs)
    s1 = _spmm_kernel(packed, g0, zeros2d)
    g1 = _rescale(s1, s1, inv)
    s2 = _spmm_kernel(packed, g1, zeros2d)
    gam8 = jnp.broadcast_to(gamma.reshape(1, D), (8, D))
    bet8 = jnp.broadcast_to(beta.reshape(1, D), (8, D))
    return _mmba(s2, s2, dis, W, gam8, bet8)


# speedup vs baseline: 1.0184x; 1.0184x over previous
import functools

import jax
import jax.numpy as jnp
from jax import lax
from jax.experimental import pallas as pl
from jax.experimental.pallas import tpu as pltpu
from jax.experimental.pallas import tpu_sc as plsc

N = 10000
D = 128
E = 320000
TAU = 2.0
V_TH = 1.0
BN_EPS = 1e-5

N_PAD = 10240
EPT = E // 32
C = 160
NWIN_FULL = 62
TAIL = EPT - NWIN_FULL * C
WPB = 8
NBLK = 8
BLKW = 2 * WPB * C
TPT = NBLK * BLKW + 2 * TAIL
CD = 1000
NWD = EPT // CD
RPT = N_PAD // 16
assert NWIN_FULL % 2 == 0 and 0 < TAIL <= C and TAIL % 8 == 0 and C % 8 == 0


def _deg_body(dst_hbm, ones_hbm, zeros_hbm, out_hbm, idx_v, ones_v, acc):
    c = lax.axis_index("c")
    s = lax.axis_index("s")
    pltpu.sync_copy(zeros_hbm.at[pl.ds(s * RPT, RPT)], acc.at[pl.ds(s * RPT, RPT)])
    pltpu.sync_copy(ones_hbm, ones_v)
    pltpu.sync_copy(dst_hbm.at[pl.ds((c * 16 + s) * EPT, EPT)], idx_v)
    plsc.subcore_barrier()

    def body(w, carry):
        pltpu.sync_copy(ones_v, acc.at[idx_v.at[pl.ds(w * CD, CD)]], add=True)
        return carry

    lax.fori_loop(0, NWD, body, 0)
    plsc.subcore_barrier()
    pltpu.sync_copy(acc.at[pl.ds(s * RPT, RPT)],
                    out_hbm.at[pl.ds(c * N_PAD + s * RPT, RPT)])


def _spmm_body(idx_hbm, g_hbm, zeros_hbm, out_hbm,
               idx_v, rows0, rows1, acc, sem0, sem1, sem_i):
    c = lax.axis_index("c")
    s = lax.axis_index("s")
    base = (c * 16 + s) * TPT
    rows = (rows0, rows1)
    sems = (sem0, sem1)

    def src_slice(w):
        p = (w // WPB) % 2
        return idx_v.at[pl.ds(p * BLKW + (w % WPB) * C, C)]

    def dst_slice(w):
        p = (w // WPB) % 2
        return idx_v.at[pl.ds(p * BLKW + WPB * C + (w % WPB) * C, C)]

    def blk_copy(w):
        blk = w // WPB
        return pltpu.make_async_copy(
            idx_hbm.at[pl.ds(base + blk * BLKW, BLKW)],
            idx_v.at[pl.ds((blk % 2) * BLKW, BLKW)], sem_i)

    def idx_gather(w, b):
        pltpu.async_copy(g_hbm.at[src_slice(w)], rows[b], sems[b])

    def gwait(w, b):
        pltpu.make_async_copy(g_hbm.at[src_slice(w)], rows[b], sems[b]).wait()

    def scat(w, b):
        pltpu.sync_copy(rows[b], acc.at[dst_slice(w)], add=True)

    @pl.when(c == 0)
    def _():
        pltpu.sync_copy(g_hbm.at[pl.ds(s * RPT, RPT)], acc.at[pl.ds(s * RPT, RPT)])

    @pl.when(c != 0)
    def _():
        pltpu.sync_copy(zeros_hbm.at[pl.ds(s * RPT, RPT)], acc.at[pl.ds(s * RPT, RPT)])

    blk_copy(0).start()
    blk_copy(0).wait()
    idx_gather(0, 0)
    plsc.subcore_barrier()

    def body(k, carry):
        w0 = 2 * k
        nb = (w0 + 2) % WPB == 0

        @pl.when(nb)
        def _():
            blk_copy(w0 + 2).start()

        gwait(w0, 0)
        idx_gather(w0 + 1, 1)
        scat(w0, 0)
        gwait(w0 + 1, 1)

        @pl.when(nb)
        def _():
            blk_copy(w0 + 2).wait()

        idx_gather(w0 + 2, 0)
        scat(w0 + 1, 1)
        return carry

    lax.fori_loop(0, NWIN_FULL // 2 - 1, body, 0)
    wl = NWIN_FULL - 2
    gwait(wl, 0)
    idx_gather(wl + 1, 1)
    scat(wl, 0)
    gwait(wl + 1, 1)
    toff = base + NBLK * BLKW
    tv = 2 * BLKW
    pltpu.sync_copy(idx_hbm.at[pl.ds(toff, 2 * TAIL)],
                    idx_v.at[pl.ds(tv, 2 * TAIL)])
    pltpu.async_copy(g_hbm.at[idx_v.at[pl.ds(tv, TAIL)]],
                     rows0.at[pl.ds(0, TAIL)], sem0)
    scat(wl + 1, 1)
    pltpu.make_async_copy(g_hbm.at[idx_v.at[pl.ds(tv, TAIL)]],
                          rows0.at[pl.ds(0, TAIL)], sem0).wait()
    pltpu.sync_copy(rows0.at[pl.ds(0, TAIL)],
                    acc.at[idx_v.at[pl.ds(tv + TAIL, TAIL)]], add=True)
    plsc.subcore_barrier()
    pltpu.sync_copy(acc.at[pl.ds(s * RPT, RPT)],
                    out_hbm.at[pl.ds(c * N_PAD + s * RPT, RPT)])


@functools.cache
def _sc_kernels():
    mesh = plsc.VectorSubcoreMesh(core_axis_name="c", subcore_axis_name="s")
    deg = pl.kernel(
        _deg_body,
        out_type=jax.ShapeDtypeStruct((2 * N_PAD,), jnp.float32),
        mesh=mesh,
        scratch_types=[
            pltpu.VMEM((EPT,), jnp.int32),
            pltpu.VMEM((CD,), jnp.float32),
            pltpu.VMEM_SHARED((N_PAD,), jnp.float32),
        ],
    )
    spmm = pl.kernel(
        _spmm_body,
        out_type=jax.ShapeDtypeStruct((2 * N_PAD, D), jnp.float32),
        mesh=mesh,
        scratch_types=[
            pltpu.VMEM((2 * BLKW + 2 * TAIL,), jnp.int32),
            pltpu.VMEM((C, D), jnp.float32),
            pltpu.VMEM((C, D), jnp.float32),
            pltpu.VMEM_SHARED((N_PAD, D), jnp.float32),
            pltpu.SemaphoreType.DMA,
            pltpu.SemaphoreType.DMA,
            pltpu.SemaphoreType.DMA,
        ],
    )
    return deg, spmm


def _pack_edges(src, dst):
    full = NWIN_FULL * C
    pad_w = NBLK * WPB - NWIN_FULL
    s2 = src.reshape(32, EPT)
    d2 = dst.reshape(32, EPT)

    def blocks(a):
        w = a[:, :full].reshape(32, NWIN_FULL, C)
        w = jnp.pad(w, ((0, 0), (0, pad_w), (0, 0)))
        return w.reshape(32, NBLK, WPB * C)

    blk = jnp.stack([blocks(s2), blocks(d2)], axis=2)
    tails = jnp.concatenate([s2[:, full:], d2[:, full:]], axis=1)
    return jnp.concatenate(
        [blk.reshape(32, NBLK * BLKW), tails], axis=1).reshape(-1)


R = 512
GB = N_PAD // R


def _prep_body(x_ref, da_ref, db_ref, g0_ref, dis_ref, inv_ref):
    deg = da_ref[...] + db_ref[...] + 1.0
    dis = jnp.where(deg > 0, lax.rsqrt(jnp.maximum(deg, 1e-12)), 0.0)
    dis_ref[...] = dis
    inv_ref[...] = dis * dis
    g0_ref[...] = x_ref[...] * dis


def _rescale_body(sa_ref, sb_ref, inv_ref, g1_ref):
    g1_ref[...] = (sa_ref[...] + sb_ref[...]) * inv_ref[...]


def _mmba_body(sa_ref, sb_ref, dis_ref, w_ref, gam_ref, bet_ref, o_ref,
               y_scr, cs_ref, cq_ref):
    i = pl.program_id(0)

    @pl.when(i == 0)
    def _():
        cs_ref[...] = jnp.zeros_like(cs_ref)
        cq_ref[...] = jnp.zeros_like(cq_ref)

    @pl.when(i < GB)
    def _():
        st = (sa_ref[...] + sb_ref[...]) * dis_ref[...]
        y = lax.dot_general(st, w_ref[...], (((1,), (1,)), ((), ())),
                            preferred_element_type=jnp.float32)
        y_scr[pl.ds(i * R, R), :] = y
        cs_ref[...] += jnp.sum(y.reshape(R // 8, 8, D), axis=0)
        cq_ref[...] += jnp.sum((y * y).reshape(R // 8, 8, D), axis=0)

    @pl.when(i >= GB)
    def _():
        j = i - GB
        invn = 1.0 / float(N)
        mean = jnp.sum(cs_ref[...], axis=0, keepdims=True) * invn
        ex2 = jnp.sum(cq_ref[...], axis=0, keepdims=True) * invn
        var = ex2 - mean * mean
        r = lax.rsqrt(var + BN_EPS)
        y = y_scr[pl.ds(j * R, R), :]
        hbn = (y - mean) * (r * gam_ref[0:1, :]) + bet_ref[0:1, :]
        v = hbn * (1.0 / TAU)
        o_ref[...] = jnp.where(v >= V_TH, 1.0, 0.0)


_blk_rd = lambda off=0: pl.BlockSpec((R, D), lambda i, o=off: (i + o, 0))
_blk_r1 = lambda off=0: pl.BlockSpec((R, 1), lambda i, o=off: (i + o, 0))
_blk_s8 = pl.BlockSpec((8, D), lambda i: (0, 0))

_prep = pl.pallas_call(
    _prep_body,
    grid=(GB,),
    in_specs=[_blk_rd(), _blk_r1(), _blk_r1(GB)],
    out_specs=[_blk_rd(), _blk_r1(), _blk_r1()],
    out_shape=[
        jax.ShapeDtypeStruct((N_PAD, D), jnp.float32),
        jax.ShapeDtypeStruct((N_PAD, 1), jnp.float32),
        jax.ShapeDtypeStruct((N_PAD, 1), jnp.float32),
    ],
)

_rescale = pl.pallas_call(
    _rescale_body,
    grid=(GB,),
    in_specs=[_blk_rd(), _blk_rd(GB), _blk_r1()],
    out_specs=_blk_rd(),
    out_shape=jax.ShapeDtypeStruct((N_PAD, D), jnp.float32),
)

_blk_in = lambda off=0: pl.BlockSpec(
    (R, D), lambda i, o=off: (jnp.minimum(i, GB - 1) + o, 0))
_blk_in1 = pl.BlockSpec((R, 1), lambda i: (jnp.minimum(i, GB - 1), 0))

_mmba = pl.pallas_call(
    _mmba_body,
    grid=(2 * GB,),
    in_specs=[_blk_in(), _blk_in(GB), _blk_in1,
              pl.BlockSpec((D, D), lambda i: (0, 0)), _blk_s8, _blk_s8],
    out_specs=pl.BlockSpec((R, D), lambda i: (jnp.maximum(i - GB, 0), 0)),
    out_shape=jax.ShapeDtypeStruct((N, D), jnp.float32),
    scratch_shapes=[
        pltpu.VMEM((N_PAD, D), jnp.float32),
        pltpu.VMEM((8, D), jnp.float32),
        pltpu.VMEM((8, D), jnp.float32),
    ],
)


def kernel(x, edge_index, W, b, gamma, beta):
    f32 = jnp.float32
    src = edge_index[0]
    dst = edge_index[1]
    x_pad = jnp.pad(x, ((0, N_PAD - N), (0, 0)))
    zeros2d = jnp.zeros((N_PAD, D), f32)
    zeros1d = jnp.zeros((N_PAD,), f32)
    ones_cd = jnp.ones((CD,), f32)

    _deg_kernel, _spmm_kernel = _sc_kernels()
    packed = _pack_edges(src, dst)
    degs = _deg_kernel(dst, ones_cd, zeros1d).reshape(2 * N_PAD, 1)
    g0, dis, inv = _prep(x_pad, degs, degs)
    s1 = _spmm_kernel(packed, g0, zeros2d)
    g1 = _rescale(s1, s1, inv)
    s2 = _spmm_kernel(packed, g1, zeros2d)
    gam8 = jnp.broadcast_to(gamma.reshape(1, D), (8, D))
    bet8 = jnp.broadcast_to(beta.reshape(1, D), (8, D))
    return _mmba(s2, s2, dis, W, gam8, bet8)


# speedup vs baseline: 1.0267x; 1.0082x over previous
import functools

import jax
import jax.numpy as jnp
from jax import lax
from jax.experimental import pallas as pl
from jax.experimental.pallas import tpu as pltpu
from jax.experimental.pallas import tpu_sc as plsc

N = 10000
D = 128
E = 320000
TAU = 2.0
V_TH = 1.0
BN_EPS = 1e-5

N_PAD = 10240
EPT = E // 32
C = 160
NWIN_FULL = 62
TAIL = EPT - NWIN_FULL * C
WPB = 8
NBLK = 8
BLKW = 2 * WPB * C
TPT = NBLK * BLKW + 2 * TAIL
CD = 1000
NWD = EPT // CD
RPT = N_PAD // 16
assert NWIN_FULL % 2 == 0 and 0 < TAIL <= C and TAIL % 8 == 0 and C % 8 == 0


def _deg_body(dst_hbm, ones_hbm, zeros_hbm, out_hbm, idx_v, ones_v, acc):
    c = lax.axis_index("c")
    s = lax.axis_index("s")
    pltpu.sync_copy(zeros_hbm.at[pl.ds(s * RPT, RPT)], acc.at[pl.ds(s * RPT, RPT)])
    pltpu.sync_copy(ones_hbm, ones_v)
    pltpu.sync_copy(dst_hbm.at[pl.ds((c * 16 + s) * EPT, EPT)], idx_v)
    plsc.subcore_barrier()

    def body(w, carry):
        pltpu.sync_copy(ones_v, acc.at[idx_v.at[pl.ds(w * CD, CD)]], add=True)
        return carry

    lax.fori_loop(0, NWD, body, 0)
    plsc.subcore_barrier()
    pltpu.sync_copy(acc.at[pl.ds(s * RPT, RPT)],
                    out_hbm.at[pl.ds(c * N_PAD + s * RPT, RPT)])


def _spmm_body(idx_hbm, g_hbm, zeros_hbm, out_hbm,
               idx_v, rows0, rows1, acc, sem0, sem1, sem_i):
    c = lax.axis_index("c")
    s = lax.axis_index("s")
    base = (c * 16 + s) * TPT
    rows = (rows0, rows1)
    sems = (sem0, sem1)

    def src_slice(w):
        p = (w // WPB) % 2
        return idx_v.at[pl.ds(p * BLKW + (w % WPB) * C, C)]

    def dst_slice(w):
        p = (w // WPB) % 2
        return idx_v.at[pl.ds(p * BLKW + WPB * C + (w % WPB) * C, C)]

    def blk_copy(w):
        blk = w // WPB
        return pltpu.make_async_copy(
            idx_hbm.at[pl.ds(base + blk * BLKW, BLKW)],
            idx_v.at[pl.ds((blk % 2) * BLKW, BLKW)], sem_i)

    def idx_gather(w, b):
        pltpu.async_copy(g_hbm.at[src_slice(w)], rows[b], sems[b])

    def gwait(w, b):
        pltpu.make_async_copy(g_hbm.at[src_slice(w)], rows[b], sems[b]).wait()

    def scat(w, b):
        pltpu.sync_copy(rows[b], acc.at[dst_slice(w)], add=True)

    @pl.when(c == 0)
    def _():
        pltpu.sync_copy(g_hbm.at[pl.ds(s * RPT, RPT)], acc.at[pl.ds(s * RPT, RPT)])

    @pl.when(c != 0)
    def _():
        pltpu.sync_copy(zeros_hbm.at[pl.ds(s * RPT, RPT)], acc.at[pl.ds(s * RPT, RPT)])

    blk_copy(0).start()
    blk_copy(0).wait()
    idx_gather(0, 0)
    plsc.subcore_barrier()

    def body(k, carry):
        w0 = 2 * k
        nb = (w0 + 2) % WPB == 0

        @pl.when(nb)
        def _():
            blk_copy(w0 + 2).start()

        gwait(w0, 0)
        idx_gather(w0 + 1, 1)
        scat(w0, 0)
        gwait(w0 + 1, 1)

        @pl.when(nb)
        def _():
            blk_copy(w0 + 2).wait()

        idx_gather(w0 + 2, 0)
        scat(w0 + 1, 1)
        return carry

    lax.fori_loop(0, NWIN_FULL // 2 - 1, body, 0)
    wl = NWIN_FULL - 2
    gwait(wl, 0)
    idx_gather(wl + 1, 1)
    scat(wl, 0)
    gwait(wl + 1, 1)
    toff = base + NBLK * BLKW
    tv = 2 * BLKW
    pltpu.sync_copy(idx_hbm.at[pl.ds(toff, 2 * TAIL)],
                    idx_v.at[pl.ds(tv, 2 * TAIL)])
    pltpu.async_copy(g_hbm.at[idx_v.at[pl.ds(tv, TAIL)]],
                     rows0.at[pl.ds(0, TAIL)], sem0)
    scat(wl + 1, 1)
    pltpu.make_async_copy(g_hbm.at[idx_v.at[pl.ds(tv, TAIL)]],
                          rows0.at[pl.ds(0, TAIL)], sem0).wait()
    pltpu.sync_copy(rows0.at[pl.ds(0, TAIL)],
                    acc.at[idx_v.at[pl.ds(tv + TAIL, TAIL)]], add=True)
    plsc.subcore_barrier()
    pltpu.sync_copy(acc.at[pl.ds(s * RPT, RPT)],
                    out_hbm.at[pl.ds(c * N_PAD + s * RPT, RPT)])


@functools.cache
def _sc_kernels():
    mesh = plsc.VectorSubcoreMesh(core_axis_name="c", subcore_axis_name="s")
    deg = pl.kernel(
        _deg_body,
        out_type=jax.ShapeDtypeStruct((2 * N_PAD,), jnp.float32),
        mesh=mesh,
        scratch_types=[
            pltpu.VMEM((EPT,), jnp.int32),
            pltpu.VMEM((CD,), jnp.float32),
            pltpu.VMEM_SHARED((N_PAD,), jnp.float32),
        ],
    )
    spmm = pl.kernel(
        _spmm_body,
        out_type=jax.ShapeDtypeStruct((2 * N_PAD, D), jnp.float32),
        mesh=mesh,
        scratch_types=[
            pltpu.VMEM((2 * BLKW + 2 * TAIL,), jnp.int32),
            pltpu.VMEM((C, D), jnp.float32),
            pltpu.VMEM((C, D), jnp.float32),
            pltpu.VMEM_SHARED((N_PAD, D), jnp.float32),
            pltpu.SemaphoreType.DMA,
            pltpu.SemaphoreType.DMA,
            pltpu.SemaphoreType.DMA,
        ],
    )
    return deg, spmm


def _pack_edges(src, dst):
    full = NWIN_FULL * C
    pad_w = NBLK * WPB - NWIN_FULL
    s2 = src.reshape(32, EPT)
    d2 = dst.reshape(32, EPT)

    def blocks(a):
        w = a[:, :full].reshape(32, NWIN_FULL, C)
        w = jnp.pad(w, ((0, 0), (0, pad_w), (0, 0)))
        return w.reshape(32, NBLK, WPB * C)

    blk = jnp.stack([blocks(s2), blocks(d2)], axis=2)
    tails = jnp.concatenate([s2[:, full:], d2[:, full:]], axis=1)
    return jnp.concatenate(
        [blk.reshape(32, NBLK * BLKW), tails], axis=1).reshape(-1)


R = 512
GB = N_PAD // R


def _prep_body(x_ref, da_ref, db_ref, g0_ref, dis_ref, inv_ref):
    i = pl.program_id(0)
    deg = da_ref[...] + db_ref[...] + 1.0
    dis = jnp.where(deg > 0, lax.rsqrt(jnp.maximum(deg, 1e-12)), 0.0)
    dis_ref[...] = dis
    inv_ref[...] = dis * dis
    row = i * R + lax.broadcasted_iota(jnp.int32, (R, 1), 0)
    g0_ref[...] = jnp.where(row < N, x_ref[...] * dis, 0.0)


def _rescale_body(sa_ref, sb_ref, inv_ref, g1_ref):
    g1_ref[...] = (sa_ref[...] + sb_ref[...]) * inv_ref[...]


def _mmba_body(sa_ref, sb_ref, dis_ref, w_ref, gam_ref, bet_ref, o_ref,
               y_scr, cs_ref, cq_ref):
    i = pl.program_id(0)

    @pl.when(i == 0)
    def _():
        cs_ref[...] = jnp.zeros_like(cs_ref)
        cq_ref[...] = jnp.zeros_like(cq_ref)

    @pl.when(i < GB)
    def _():
        st = (sa_ref[...] + sb_ref[...]) * dis_ref[...]
        y = lax.dot_general(st, w_ref[...], (((1,), (1,)), ((), ())),
                            preferred_element_type=jnp.float32)
        y_scr[pl.ds(i * R, R), :] = y
        cs_ref[...] += jnp.sum(y.reshape(R // 8, 8, D), axis=0)
        cq_ref[...] += jnp.sum((y * y).reshape(R // 8, 8, D), axis=0)

    @pl.when(i >= GB)
    def _():
        j = i - GB
        invn = 1.0 / float(N)
        mean = jnp.sum(cs_ref[...], axis=0, keepdims=True) * invn
        ex2 = jnp.sum(cq_ref[...], axis=0, keepdims=True) * invn
        var = ex2 - mean * mean
        r = lax.rsqrt(var + BN_EPS)
        y = y_scr[pl.ds(j * R, R), :]
        hbn = (y - mean) * (r * gam_ref[0:1, :]) + bet_ref[0:1, :]
        v = hbn * (1.0 / TAU)
        o_ref[...] = jnp.where(v >= V_TH, 1.0, 0.0)


_blk_rd = lambda off=0: pl.BlockSpec((R, D), lambda i, o=off: (i + o, 0))
_blk_r1 = lambda off=0: pl.BlockSpec((R, 1), lambda i, o=off: (i + o, 0))
_blk_s8 = pl.BlockSpec((8, D), lambda i: (0, 0))

_prep = pl.pallas_call(
    _prep_body,
    grid=(GB,),
    in_specs=[_blk_rd(), _blk_r1(), _blk_r1(GB)],
    out_specs=[_blk_rd(), _blk_r1(), _blk_r1()],
    out_shape=[
        jax.ShapeDtypeStruct((N_PAD, D), jnp.float32),
        jax.ShapeDtypeStruct((N_PAD, 1), jnp.float32),
        jax.ShapeDtypeStruct((N_PAD, 1), jnp.float32),
    ],
)

_rescale = pl.pallas_call(
    _rescale_body,
    grid=(GB,),
    in_specs=[_blk_rd(), _blk_rd(GB), _blk_r1()],
    out_specs=_blk_rd(),
    out_shape=jax.ShapeDtypeStruct((N_PAD, D), jnp.float32),
)

_blk_in = lambda off=0: pl.BlockSpec(
    (R, D), lambda i, o=off: (jnp.minimum(i, GB - 1) + o, 0))
_blk_in1 = pl.BlockSpec((R, 1), lambda i: (jnp.minimum(i, GB - 1), 0))

_mmba = pl.pallas_call(
    _mmba_body,
    grid=(2 * GB,),
    in_specs=[_blk_in(), _blk_in(GB), _blk_in1,
              pl.BlockSpec((D, D), lambda i: (0, 0)), _blk_s8, _blk_s8],
    out_specs=pl.BlockSpec((R, D), lambda i: (jnp.maximum(i - GB, 0), 0)),
    out_shape=jax.ShapeDtypeStruct((N, D), jnp.float32),
    scratch_shapes=[
        pltpu.VMEM((N_PAD, D), jnp.float32),
        pltpu.VMEM((8, D), jnp.float32),
        pltpu.VMEM((8, D), jnp.float32),
    ],
)


def kernel(x, edge_index, W, b, gamma, beta):
    f32 = jnp.float32
    src = edge_index[0]
    dst = edge_index[1]
    zeros2d = jnp.zeros((N_PAD, D), f32)
    zeros1d = jnp.zeros((N_PAD,), f32)
    ones_cd = jnp.ones((CD,), f32)

    _deg_kernel, _spmm_kernel = _sc_kernels()
    packed = _pack_edges(src, dst)
    degs = _deg_kernel(dst, ones_cd, zeros1d).reshape(2 * N_PAD, 1)
    g0, dis, inv = _prep(x, degs, degs)
    s1 = _spmm_kernel(packed, g0, zeros2d)
    g1 = _rescale(s1, s1, inv)
    s2 = _spmm_kernel(packed, g1, zeros2d)
    gam8 = jnp.broadcast_to(gamma.reshape(1, D), (8, D))
    bet8 = jnp.broadcast_to(beta.reshape(1, D), (8, D))
    return _mmba(s2, s2, dis, W, gam8, bet8)
